# baseline (speedup 1.0000x reference)
import functools

import jax
import jax.numpy as jnp
from jax import lax
from jax.experimental import pallas as pl
from jax.experimental.pallas import tpu as pltpu
from jax.experimental.pallas import tpu_sc as plsc

N = 10000
E = 320000
F = 128
NC = 2
NS = 16
NW = NC * NS
EW = E // NW
CH = 40
NCHUNK = EW // CH
RTA = 624
RTB = N - (NS - 1) * RTA

_MESH = plsc.VectorSubcoreMesh(
    core_axis_name="c", subcore_axis_name="s", num_cores=NC, num_subcores=NS)


def _deg_body(dst3_hbm, zeros_hbm, out_hbm, didx, ones_v, acc, ssem):
    c = lax.axis_index("c")
    s = lax.axis_index("s")
    wid = s * NC + c
    for off in (0, 16, 24):
        ones_v[pl.ds(off, 16)] = jnp.ones((16,), jnp.float32)
    @pl.when(s == 0)
    def _():
        pltpu.sync_copy(zeros_hbm, acc)
    pltpu.sync_copy(dst3_hbm.at[wid], didx)
    plsc.subcore_barrier()

    def body(j, carry):
        base = j * NBUF
        for b in range(NBUF):
            pltpu.async_copy(ones_v, acc.at[didx.at[base + b]], ssem,
                             add=True)
        for b in range(NBUF):
            pltpu.make_async_copy(ones_v, acc.at[didx.at[base + b]],
                                  ssem).wait()
        return carry
    lax.fori_loop(0, NB, body, 0)
    plsc.subcore_barrier()

    @pl.when(s == 0)
    def _():
        pltpu.sync_copy(acc, out_hbm.at[c])


def _deg_call(dst3, zeros1d):
    k = pl.kernel(
        _deg_body,
        out_type=jax.ShapeDtypeStruct((NC, N), jnp.float32),
        mesh=_MESH,
        scratch_types=[
            pltpu.VMEM((NCHUNK, CH), jnp.int32),
            pltpu.VMEM((CH,), jnp.float32),
            pltpu.VMEM_SHARED((N,), jnp.float32),
            pltpu.SemaphoreType.DMA,
        ],
    )
    return k(dst3, zeros1d)


NBUF = 2
NB = NCHUNK // NBUF


def _scatter_body(x_hbm, src3_hbm, dst3_hbm, zeros_hbm, out_hbm,
                  sidx, didx, rows, acc, isem, gsem, ssem):
    c = lax.axis_index("c")
    s = lax.axis_index("s")
    wid = s * NC + c
    @pl.when(s < NS - 1)
    def _():
        o = pl.multiple_of(s * RTA, 8)
        pltpu.sync_copy(zeros_hbm.at[pl.ds(o, RTA)], acc.at[pl.ds(o, RTA)])
    @pl.when(s == NS - 1)
    def _():
        pltpu.sync_copy(zeros_hbm.at[pl.ds((NS - 1) * RTA, RTB)],
                        acc.at[pl.ds((NS - 1) * RTA, RTB)])

    def idx_start(i, q):
        pltpu.async_copy(src3_hbm.at[wid, i], sidx.at[q], isem.at[q])
        pltpu.async_copy(dst3_hbm.at[wid, i], didx.at[q], isem.at[q])

    def idx_wait(i, q):
        pltpu.make_async_copy(src3_hbm.at[wid, i], sidx.at[q],
                              isem.at[q]).wait()
        pltpu.make_async_copy(dst3_hbm.at[wid, i], didx.at[q],
                              isem.at[q]).wait()

    def scat_wait(r):
        pltpu.make_async_copy(rows.at[r], acc.at[didx.at[0]],
                              ssem.at[r]).wait()

    def chunk(i, r, q):
        if isinstance(i, int):
            if i >= 3:
                scat_wait(r)
            if i + 3 < NCHUNK:
                idx_start(i + 3, (q + 3) % 6)
        else:
            @pl.when(i >= 3)
            def _():
                scat_wait(r)
            @pl.when(i + 3 < NCHUNK)
            def _():
                idx_start(i + 3, (q + 3) % 6)
        idx_wait(i, q)
        pltpu.async_copy(x_hbm.at[sidx.at[q]], rows.at[r], gsem.at[r])
        pltpu.make_async_copy(x_hbm.at[sidx.at[q]], rows.at[r],
                              gsem.at[r]).wait()
        pltpu.async_copy(rows.at[r], acc.at[didx.at[q]], ssem.at[r],
                         add=True)

    idx_start(0, 0)
    idx_start(1, 1)
    idx_start(2, 2)
    plsc.subcore_barrier()

    def body(t, carry):
        for b in range(6):
            chunk(6 * t + b, b % 3, b)
        return carry
    lax.fori_loop(0, NCHUNK // 6, body, 0)
    chunk(246, 0, 0)
    chunk(247, 1, 1)
    chunk(248, 2, 2)
    chunk(249, 0, 3)
    scat_wait(1)
    scat_wait(2)
    scat_wait(0)
    plsc.subcore_barrier()

    @pl.when(s < NS - 1)
    def _():
        o = pl.multiple_of(s * RTA, 8)
        pltpu.sync_copy(acc.at[pl.ds(o, RTA)], out_hbm.at[c, pl.ds(o, RTA)])
    @pl.when(s == NS - 1)
    def _():
        pltpu.sync_copy(acc.at[pl.ds((NS - 1) * RTA, RTB)],
                        out_hbm.at[c, pl.ds((NS - 1) * RTA, RTB)])


def _scatter_call(x, src3, dst3, zeros2d):
    k = pl.kernel(
        _scatter_body,
        out_type=jax.ShapeDtypeStruct((NC, N, F), jnp.float32),
        mesh=_MESH,
        scratch_types=[
            pltpu.VMEM((6, CH), jnp.int32),
            pltpu.VMEM((6, CH), jnp.int32),
            pltpu.VMEM((3, CH, F), jnp.float32),
            pltpu.VMEM_SHARED((N, F), jnp.float32),
            pltpu.SemaphoreType.DMA((6,)),
            pltpu.SemaphoreType.DMA((3,)),
            pltpu.SemaphoreType.DMA((3,)),
        ],
    )
    return k(x, src3, dst3, zeros2d)


_R = 2000
_G = N // _R


def _tc1_body(x_ref, w1_ref, b1_ref, w2_ref, b2_ref, dinv_ref, h_ref, u_ref):
    x = x_ref[...]
    h = jnp.maximum(jnp.dot(x, w1_ref[...],
                            preferred_element_type=jnp.float32) + b1_ref[...], 0.0)
    h = jnp.maximum(jnp.dot(h, w2_ref[...],
                            preferred_element_type=jnp.float32) + b2_ref[...], 0.0)
    h_ref[...] = h
    u_ref[...] = h * dinv_ref[...]


def _tc1_call(x, W1, b1, W2, b2, dinvc):
    full = lambda shp: pl.BlockSpec(shp, lambda i: (0,) * len(shp))
    return pl.pallas_call(
        _tc1_body,
        grid=(_G,),
        in_specs=[
            pl.BlockSpec((_R, F), lambda i: (i, 0)),
            full((F, F)), full((F,)), full((F, F)), full((F,)),
            pl.BlockSpec((_R, 1), lambda i: (i, 0)),
        ],
        out_specs=[pl.BlockSpec((_R, F), lambda i: (i, 0))] * 2,
        out_shape=[jax.ShapeDtypeStruct((N, F), jnp.float32)] * 2,
    )(x, W1, b1, W2, b2, dinvc)


def _tc2_body(s1_ref, dinv_ref, dinv2_ref, b_ref, v_ref):
    s1 = s1_ref[0] + s1_ref[1]
    b_ref[...] = s1 * dinv_ref[...]
    v_ref[...] = s1 * dinv2_ref[...]


def _tc2_call(s1p, dinvc, dinv2c):
    return pl.pallas_call(
        _tc2_body,
        grid=(_G,),
        in_specs=[
            pl.BlockSpec((NC, _R, F), lambda i: (0, i, 0)),
            pl.BlockSpec((_R, 1), lambda i: (i, 0)),
            pl.BlockSpec((_R, 1), lambda i: (i, 0)),
        ],
        out_specs=[pl.BlockSpec((_R, F), lambda i: (i, 0))] * 2,
        out_shape=[jax.ShapeDtypeStruct((N, F), jnp.float32)] * 2,
    )(s1p, dinvc, dinv2c)


def _tc3_body(h_ref, b_ref, s2_ref, dinv_ref, diag1_ref, wl1_ref, bl1_ref,
              bl2_ref, bl3_ref, diag2_ref, diag3_ref, w3_ref, b3_ref,
              w4_ref, b4_ref, logits_ref, hh_ref):
    h = h_ref[...]
    B = b_ref[...]
    C = (s2_ref[0] + s2_ref[1]) * dinv_ref[...]
    diag1 = diag1_ref[...]
    diag2 = diag2_ref[...]
    diag3 = diag3_ref[...]

    def f12(dg):
        d1, d2 = dg[1], dg[2]
        f1 = h - B * d1
        f2 = h - B * (d1 + d2) + C * (d1 * d2)
        return f1, f2

    f1_1, f2_1 = f12(diag1)
    h1 = (jnp.dot(3.0 * h * diag1[0], wl1_ref[...],
                  preferred_element_type=jnp.float32) + bl1_ref[...]
          - 3.0 * f1_1 + 0.75 * f2_1)
    f1_2, f2_2 = f12(diag2)
    h2 = bl2_ref[...] + 3.0 * f1_2 - 1.5 * f2_2
    _, f2_3 = f12(diag3)
    h3 = bl3_ref[...] + 0.75 * f2_3

    w3 = w3_ref[...]
    hh = (jnp.dot(h1, w3[0:F], preferred_element_type=jnp.float32)
          + jnp.dot(h2, w3[F:2 * F], preferred_element_type=jnp.float32)
          + jnp.dot(h3, w3[2 * F:3 * F], preferred_element_type=jnp.float32)
          + b3_ref[...])
    hh = jnp.maximum(hh, 0.0)
    hh_ref[...] = hh
    logits_ref[...] = jnp.dot(hh, w4_ref[...],
                              preferred_element_type=jnp.float32) + b4_ref[...]


def _tc3_call(h, Bmat, s2p, dinvc, diag1, Wl1, bl1, bl2, bl3, diag2, diag3,
              W3, b3, W4, b4):
    full = lambda shp: pl.BlockSpec(shp, lambda i: (0,) * len(shp))
    return pl.pallas_call(
        _tc3_body,
        grid=(_G,),
        in_specs=[
            pl.BlockSpec((_R, F), lambda i: (i, 0)),
            pl.BlockSpec((_R, F), lambda i: (i, 0)),
            pl.BlockSpec((NC, _R, F), lambda i: (0, i, 0)),
            pl.BlockSpec((_R, 1), lambda i: (i, 0)),
            full((3, F)), full((F, F)), full((F,)), full((F,)), full((F,)),
            full((3, F)), full((3, F)), full((3 * F, F)), full((F,)),
            full((F, 2)), full((2,)),
        ],
        out_specs=[pl.BlockSpec((_R, 2), lambda i: (i, 0)),
                   pl.BlockSpec((_R, F), lambda i: (i, 0))],
        out_shape=[jax.ShapeDtypeStruct((N, 2), jnp.float32),
                   jax.ShapeDtypeStruct((N, F), jnp.float32)],
    )(h, Bmat, s2p, dinvc, diag1, Wl1, bl1, bl2, bl3, diag2, diag3,
      W3, b3, W4, b4)


def kernel(in_feat, edge_index, W1, b1, W2, b2, diag1, Wl1, bl1,
           diag2, Wl2, bl2, diag3, Wl3, bl3, W3, b3, W4, b4):
    src3 = edge_index[0].reshape(NW, NCHUNK, CH)
    dst3 = edge_index[1].reshape(NW, NCHUNK, CH)
    zeros1d = jnp.zeros((N,), jnp.float32)
    zeros2d = jnp.zeros((N, F), jnp.float32)

    degp = _deg_call(dst3, zeros1d)
    deg = degp[0] + degp[1]
    dinv = lax.rsqrt(jnp.maximum(deg, 1.0))
    dinvc = dinv[:, None]
    dinv2c = (dinv * dinv)[:, None]

    h, u = _tc1_call(in_feat, W1, b1, W2, b2, dinvc)
    s1p = _scatter_call(u, src3, dst3, zeros2d)
    Bmat, v = _tc2_call(s1p, dinvc, dinv2c)
    s2p = _scatter_call(v, src3, dst3, zeros2d)
    logits, hh = _tc3_call(h, Bmat, s2p, dinvc, diag1, Wl1, bl1, bl2, bl3,
                           diag2, diag3, W3, b3, W4, b4)
    return (logits, hh)


# speedup vs baseline: 1.3845x; 1.3845x over previous
import functools

import jax
import jax.numpy as jnp
from jax import lax
from jax.experimental import pallas as pl
from jax.experimental.pallas import tpu as pltpu
from jax.experimental.pallas import tpu_sc as plsc

N = 10000
E = 320000
F = 128
NC = 2
NS = 16
NW = NC * NS
EW = E // NW
CH = 40
NCHUNK = EW // CH
RTA = 624
RTB = N - (NS - 1) * RTA

_MESH = plsc.VectorSubcoreMesh(
    core_axis_name="c", subcore_axis_name="s", num_cores=NC, num_subcores=NS)


def _deg_body(dst3_hbm, zeros_hbm, out_hbm, didx, ones_v, acc, isem, ssem):
    c = lax.axis_index("c")
    s = lax.axis_index("s")
    wid = s * NC + c
    for off in (0, 16, 24):
        ones_v[pl.ds(off, 16)] = jnp.ones((16,), jnp.float32)
    @pl.when(s == 0)
    def _():
        pltpu.sync_copy(zeros_hbm, acc)

    def idx_start(i, q):
        pltpu.async_copy(dst3_hbm.at[wid, i], didx.at[q], isem.at[q])

    def idx_wait(i, q):
        pltpu.make_async_copy(dst3_hbm.at[wid, i], didx.at[q],
                              isem.at[q]).wait()

    def scat_wait(q):
        pltpu.make_async_copy(ones_v, acc.at[didx.at[q]], ssem.at[q]).wait()

    def chunk(i, q):
        if isinstance(i, int):
            if i >= 2:
                scat_wait((q + 2) % 4)
            if i + 2 < NCHUNK:
                idx_start(i + 2, (q + 2) % 4)
        else:
            @pl.when(i >= 2)
            def _():
                scat_wait((q + 2) % 4)
            @pl.when(i + 2 < NCHUNK)
            def _():
                idx_start(i + 2, (q + 2) % 4)
        idx_wait(i, q)
        pltpu.async_copy(ones_v, acc.at[didx.at[q]], ssem.at[q], add=True)

    idx_start(0, 0)
    idx_start(1, 1)
    plsc.subcore_barrier()

    def body(t, carry):
        for b in range(4):
            chunk(4 * t + b, b)
        return carry
    lax.fori_loop(0, NCHUNK // 4, body, 0)
    chunk(NCHUNK - 2, 0)
    chunk(NCHUNK - 1, 1)
    scat_wait(0)
    scat_wait(1)
    plsc.subcore_barrier()

    @pl.when(s == 0)
    def _():
        pltpu.sync_copy(acc, out_hbm.at[c])


def _deg_call(dst3, zeros1d):
    k = pl.kernel(
        _deg_body,
        out_type=jax.ShapeDtypeStruct((NC, N), jnp.float32),
        mesh=_MESH,
        scratch_types=[
            pltpu.VMEM((4, CH), jnp.int32),
            pltpu.VMEM((CH,), jnp.float32),
            pltpu.VMEM_SHARED((N,), jnp.float32),
            pltpu.SemaphoreType.DMA((4,)),
            pltpu.SemaphoreType.DMA((4,)),
        ],
    )
    return k(dst3, zeros1d)


NBUF = 2
NB = NCHUNK // NBUF


def _scatter_body(x_hbm, src3_hbm, dst3_hbm, zeros_hbm, out_hbm,
                  sidx, didx, rows, acc, isem, gsem, ssem):
    c = lax.axis_index("c")
    s = lax.axis_index("s")
    wid = s * NC + c
    @pl.when(s < NS - 1)
    def _():
        o = pl.multiple_of(s * RTA, 8)
        pltpu.sync_copy(zeros_hbm.at[pl.ds(o, RTA)], acc.at[pl.ds(o, RTA)])
    @pl.when(s == NS - 1)
    def _():
        pltpu.sync_copy(zeros_hbm.at[pl.ds((NS - 1) * RTA, RTB)],
                        acc.at[pl.ds((NS - 1) * RTA, RTB)])

    def idx_start(i, q):
        pltpu.async_copy(src3_hbm.at[wid, i], sidx.at[q], isem.at[q])
        pltpu.async_copy(dst3_hbm.at[wid, i], didx.at[q], isem.at[q])

    def idx_wait(i, q):
        pltpu.make_async_copy(src3_hbm.at[wid, i], sidx.at[q],
                              isem.at[q]).wait()
        pltpu.make_async_copy(dst3_hbm.at[wid, i], didx.at[q],
                              isem.at[q]).wait()

    def gat_start(i, r, q):
        pltpu.async_copy(x_hbm.at[sidx.at[q]], rows.at[r], gsem.at[r])

    def gat_wait(r, q):
        pltpu.make_async_copy(x_hbm.at[sidx.at[q]], rows.at[r],
                              gsem.at[r]).wait()

    def scat_wait(r):
        pltpu.make_async_copy(rows.at[r], acc.at[didx.at[0]],
                              ssem.at[r]).wait()

    def chunk(i, r, q):
        gat_wait(r, q)
        pltpu.async_copy(rows.at[r], acc.at[didx.at[q]], ssem.at[r],
                         add=True)
        if isinstance(i, int):
            if i >= 2:
                scat_wait((r + 2) % 4)
            if i + 4 < NCHUNK:
                idx_start(i + 4, (q + 4) % 6)
            if i + 2 < NCHUNK:
                idx_wait(i + 2, (q + 2) % 6)
                gat_start(i + 2, (r + 2) % 4, (q + 2) % 6)
        else:
            @pl.when(i >= 2)
            def _():
                scat_wait((r + 2) % 4)
            idx_start(i + 4, (q + 4) % 6)
            idx_wait(i + 2, (q + 2) % 6)
            gat_start(i + 2, (r + 2) % 4, (q + 2) % 6)

    for j in range(4):
        idx_start(j, j)
    plsc.subcore_barrier()
    idx_wait(0, 0)
    gat_start(0, 0, 0)
    idx_wait(1, 1)
    gat_start(1, 1, 1)

    def body(t, carry):
        for b in range(12):
            chunk(12 * t + b, b % 4, b % 6)
        return carry
    lax.fori_loop(0, (NCHUNK - 10) // 12, body, 0)
    for i in range(NCHUNK - 10, NCHUNK):
        chunk(i, i % 4, i % 6)
    scat_wait((NCHUNK - 2) % 4)
    scat_wait((NCHUNK - 1) % 4)
    plsc.subcore_barrier()

    @pl.when(s < NS - 1)
    def _():
        o = pl.multiple_of(s * RTA, 8)
        pltpu.sync_copy(acc.at[pl.ds(o, RTA)], out_hbm.at[c, pl.ds(o, RTA)])
    @pl.when(s == NS - 1)
    def _():
        pltpu.sync_copy(acc.at[pl.ds((NS - 1) * RTA, RTB)],
                        out_hbm.at[c, pl.ds((NS - 1) * RTA, RTB)])


def _scatter_call(x, src3, dst3, zeros2d):
    k = pl.kernel(
        _scatter_body,
        out_type=jax.ShapeDtypeStruct((NC, N, F), jnp.float32),
        mesh=_MESH,
        scratch_types=[
            pltpu.VMEM((6, CH), jnp.int32),
            pltpu.VMEM((6, CH), jnp.int32),
            pltpu.VMEM((4, CH, F), jnp.float32),
            pltpu.VMEM_SHARED((N, F), jnp.float32),
            pltpu.SemaphoreType.DMA((6,)),
            pltpu.SemaphoreType.DMA((4,)),
            pltpu.SemaphoreType.DMA((4,)),
        ],
    )
    return k(x, src3, dst3, zeros2d)


_R = 2000
_G = N // _R


def _tc1_body(x_ref, w1_ref, b1_ref, w2_ref, b2_ref, dinv_ref, h_ref, u_ref):
    x = x_ref[...]
    h = jnp.maximum(jnp.dot(x, w1_ref[...],
                            preferred_element_type=jnp.float32) + b1_ref[...], 0.0)
    h = jnp.maximum(jnp.dot(h, w2_ref[...],
                            preferred_element_type=jnp.float32) + b2_ref[...], 0.0)
    h_ref[...] = h
    u_ref[...] = h * dinv_ref[...]


def _tc1_call(x, W1, b1, W2, b2, dinvc):
    full = lambda shp: pl.BlockSpec(shp, lambda i: (0,) * len(shp))
    return pl.pallas_call(
        _tc1_body,
        grid=(_G,),
        in_specs=[
            pl.BlockSpec((_R, F), lambda i: (i, 0)),
            full((F, F)), full((F,)), full((F, F)), full((F,)),
            pl.BlockSpec((_R, 1), lambda i: (i, 0)),
        ],
        out_specs=[pl.BlockSpec((_R, F), lambda i: (i, 0))] * 2,
        out_shape=[jax.ShapeDtypeStruct((N, F), jnp.float32)] * 2,
    )(x, W1, b1, W2, b2, dinvc)


def _tc2_body(s1_ref, dinv_ref, dinv2_ref, b_ref, v_ref):
    s1 = s1_ref[0] + s1_ref[1]
    b_ref[...] = s1 * dinv_ref[...]
    v_ref[...] = s1 * dinv2_ref[...]


def _tc2_call(s1p, dinvc, dinv2c):
    return pl.pallas_call(
        _tc2_body,
        grid=(_G,),
        in_specs=[
            pl.BlockSpec((NC, _R, F), lambda i: (0, i, 0)),
            pl.BlockSpec((_R, 1), lambda i: (i, 0)),
            pl.BlockSpec((_R, 1), lambda i: (i, 0)),
        ],
        out_specs=[pl.BlockSpec((_R, F), lambda i: (i, 0))] * 2,
        out_shape=[jax.ShapeDtypeStruct((N, F), jnp.float32)] * 2,
    )(s1p, dinvc, dinv2c)


def _tc3_body(h_ref, b_ref, s2_ref, dinv_ref, diag1_ref, wl1_ref, bl1_ref,
              bl2_ref, bl3_ref, diag2_ref, diag3_ref, w3_ref, b3_ref,
              w4_ref, b4_ref, logits_ref, hh_ref):
    h = h_ref[...]
    B = b_ref[...]
    C = (s2_ref[0] + s2_ref[1]) * dinv_ref[...]
    diag1 = diag1_ref[...]
    diag2 = diag2_ref[...]
    diag3 = diag3_ref[...]

    def f12(dg):
        d1, d2 = dg[1], dg[2]
        f1 = h - B * d1
        f2 = h - B * (d1 + d2) + C * (d1 * d2)
        return f1, f2

    f1_1, f2_1 = f12(diag1)
    h1 = (jnp.dot(3.0 * h * diag1[0], wl1_ref[...],
                  preferred_element_type=jnp.float32) + bl1_ref[...]
          - 3.0 * f1_1 + 0.75 * f2_1)
    f1_2, f2_2 = f12(diag2)
    h2 = bl2_ref[...] + 3.0 * f1_2 - 1.5 * f2_2
    _, f2_3 = f12(diag3)
    h3 = bl3_ref[...] + 0.75 * f2_3

    w3 = w3_ref[...]
    hh = (jnp.dot(h1, w3[0:F], preferred_element_type=jnp.float32)
          + jnp.dot(h2, w3[F:2 * F], preferred_element_type=jnp.float32)
          + jnp.dot(h3, w3[2 * F:3 * F], preferred_element_type=jnp.float32)
          + b3_ref[...])
    hh = jnp.maximum(hh, 0.0)
    hh_ref[...] = hh
    logits_ref[...] = jnp.dot(hh, w4_ref[...],
                              preferred_element_type=jnp.float32) + b4_ref[...]


def _tc3_call(h, Bmat, s2p, dinvc, diag1, Wl1, bl1, bl2, bl3, diag2, diag3,
              W3, b3, W4, b4):
    full = lambda shp: pl.BlockSpec(shp, lambda i: (0,) * len(shp))
    return pl.pallas_call(
        _tc3_body,
        grid=(_G,),
        in_specs=[
            pl.BlockSpec((_R, F), lambda i: (i, 0)),
            pl.BlockSpec((_R, F), lambda i: (i, 0)),
            pl.BlockSpec((NC, _R, F), lambda i: (0, i, 0)),
            pl.BlockSpec((_R, 1), lambda i: (i, 0)),
            full((3, F)), full((F, F)), full((F,)), full((F,)), full((F,)),
            full((3, F)), full((3, F)), full((3 * F, F)), full((F,)),
            full((F, 2)), full((2,)),
        ],
        out_specs=[pl.BlockSpec((_R, 2), lambda i: (i, 0)),
                   pl.BlockSpec((_R, F), lambda i: (i, 0))],
        out_shape=[jax.ShapeDtypeStruct((N, 2), jnp.float32),
                   jax.ShapeDtypeStruct((N, F), jnp.float32)],
    )(h, Bmat, s2p, dinvc, diag1, Wl1, bl1, bl2, bl3, diag2, diag3,
      W3, b3, W4, b4)


def kernel(in_feat, edge_index, W1, b1, W2, b2, diag1, Wl1, bl1,
           diag2, Wl2, bl2, diag3, Wl3, bl3, W3, b3, W4, b4):
    src3 = edge_index[0].reshape(NW, NCHUNK, CH)
    dst3 = edge_index[1].reshape(NW, NCHUNK, CH)
    zeros1d = jnp.zeros((N,), jnp.float32)
    zeros2d = jnp.zeros((N, F), jnp.float32)

    degp = _deg_call(dst3, zeros1d)
    deg = degp[0] + degp[1]
    dinv = lax.rsqrt(jnp.maximum(deg, 1.0))
    dinvc = dinv[:, None]
    dinv2c = (dinv * dinv)[:, None]

    h, u = _tc1_call(in_feat, W1, b1, W2, b2, dinvc)
    s1p = _scatter_call(u, src3, dst3, zeros2d)
    Bmat, v = _tc2_call(s1p, dinvc, dinv2c)
    s2p = _scatter_call(v, src3, dst3, zeros2d)
    logits, hh = _tc3_call(h, Bmat, s2p, dinvc, diag1, Wl1, bl1, bl2, bl3,
                           diag2, diag3, W3, b3, W4, b4)
    return (logits, hh)


# speedup vs baseline: 1.6636x; 1.2016x over previous
import functools

import jax
import jax.numpy as jnp
from jax import lax
from jax.experimental import pallas as pl
from jax.experimental.pallas import tpu as pltpu
from jax.experimental.pallas import tpu_sc as plsc

N = 10000
E = 320000
F = 128
NC = 2
NS = 16
NW = NC * NS
EW = E // NW
CH = 40
NCHUNK = EW // CH
RTA = 624
RTB = N - (NS - 1) * RTA

_MESH = plsc.VectorSubcoreMesh(
    core_axis_name="c", subcore_axis_name="s", num_cores=NC, num_subcores=NS)


def _deg_body(dst3_hbm, zeros_hbm, out_hbm, didx, ones_v, acc, isem, ssem):
    c = lax.axis_index("c")
    s = lax.axis_index("s")
    wid = s * NC + c
    for off in (0, 16, 24):
        ones_v[pl.ds(off, 16)] = jnp.ones((16,), jnp.float32)
    @pl.when(s == 0)
    def _():
        pltpu.sync_copy(zeros_hbm, acc)

    def idx_start(i, q):
        pltpu.async_copy(dst3_hbm.at[wid, i], didx.at[q], isem.at[q])

    def idx_wait(i, q):
        pltpu.make_async_copy(dst3_hbm.at[wid, i], didx.at[q],
                              isem.at[q]).wait()

    def scat_wait(q):
        pltpu.make_async_copy(ones_v, acc.at[didx.at[q]], ssem.at[q]).wait()

    def chunk(i, q):
        if isinstance(i, int):
            if i >= 2:
                scat_wait((q + 2) % 4)
            if i + 2 < NCHUNK:
                idx_start(i + 2, (q + 2) % 4)
        else:
            @pl.when(i >= 2)
            def _():
                scat_wait((q + 2) % 4)
            @pl.when(i + 2 < NCHUNK)
            def _():
                idx_start(i + 2, (q + 2) % 4)
        idx_wait(i, q)
        pltpu.async_copy(ones_v, acc.at[didx.at[q]], ssem.at[q], add=True)

    idx_start(0, 0)
    idx_start(1, 1)
    plsc.subcore_barrier()

    def body(t, carry):
        for b in range(4):
            chunk(4 * t + b, b)
        return carry
    lax.fori_loop(0, NCHUNK // 4, body, 0)
    chunk(NCHUNK - 2, 0)
    chunk(NCHUNK - 1, 1)
    scat_wait(0)
    scat_wait(1)
    plsc.subcore_barrier()

    @pl.when(s == 0)
    def _():
        pltpu.sync_copy(acc, out_hbm.at[c])


def _deg_call(dst3, zeros1d):
    k = pl.kernel(
        _deg_body,
        out_type=jax.ShapeDtypeStruct((NC, N), jnp.float32),
        mesh=_MESH,
        scratch_types=[
            pltpu.VMEM((4, CH), jnp.int32),
            pltpu.VMEM((CH,), jnp.float32),
            pltpu.VMEM_SHARED((N,), jnp.float32),
            pltpu.SemaphoreType.DMA((4,)),
            pltpu.SemaphoreType.DMA((4,)),
        ],
    )
    return k(dst3, zeros1d)


NBUF = 2
NB = NCHUNK // NBUF


def _scatter_body(x_hbm, src3_hbm, dst3_hbm, zeros_hbm, out_hbm,
                  sidx, didx, rows, acc, isem, gsem, ssem):
    c = lax.axis_index("c")
    s = lax.axis_index("s")
    wid = s * NC + c
    @pl.when(s < NS - 1)
    def _():
        o = pl.multiple_of(s * RTA, 8)
        pltpu.sync_copy(zeros_hbm.at[pl.ds(o, RTA)], acc.at[pl.ds(o, RTA)])
    @pl.when(s == NS - 1)
    def _():
        pltpu.sync_copy(zeros_hbm.at[pl.ds((NS - 1) * RTA, RTB)],
                        acc.at[pl.ds((NS - 1) * RTA, RTB)])

    def idx_start(i, q):
        pltpu.async_copy(src3_hbm.at[wid, i], sidx.at[q], isem.at[q])
        pltpu.async_copy(dst3_hbm.at[wid, i], didx.at[q], isem.at[q])

    def idx_wait(i, q):
        pltpu.make_async_copy(src3_hbm.at[wid, i], sidx.at[q],
                              isem.at[q]).wait()
        pltpu.make_async_copy(dst3_hbm.at[wid, i], didx.at[q],
                              isem.at[q]).wait()

    def gat_start(i, r, q):
        pltpu.async_copy(x_hbm.at[sidx.at[q]], rows.at[r], gsem.at[r])

    def gat_wait(r, q):
        pltpu.make_async_copy(x_hbm.at[sidx.at[q]], rows.at[r],
                              gsem.at[r]).wait()

    def scat_wait(r):
        pltpu.make_async_copy(rows.at[r], acc.at[didx.at[0]],
                              ssem.at[r]).wait()

    def chunk(i, r, q):
        gat_wait(r, q)
        pltpu.async_copy(rows.at[r], acc.at[didx.at[q]], ssem.at[r],
                         add=True)
        if isinstance(i, int):
            if i >= 1:
                scat_wait((r + 3) % 4)
            if i + 5 < NCHUNK:
                idx_start(i + 5, (q + 5) % 6)
            if i + 3 < NCHUNK:
                idx_wait(i + 3, (q + 3) % 6)
                gat_start(i + 3, (r + 3) % 4, (q + 3) % 6)
        else:
            @pl.when(i >= 1)
            def _():
                scat_wait((r + 3) % 4)
            idx_start(i + 5, (q + 5) % 6)
            idx_wait(i + 3, (q + 3) % 6)
            gat_start(i + 3, (r + 3) % 4, (q + 3) % 6)

    for j in range(5):
        idx_start(j, j)
    plsc.subcore_barrier()
    idx_wait(0, 0)
    gat_start(0, 0, 0)
    idx_wait(1, 1)
    gat_start(1, 1, 1)
    idx_wait(2, 2)
    gat_start(2, 2, 2)

    def body(t, carry):
        for b in range(12):
            chunk(12 * t + b, b % 4, b % 6)
        return carry
    lax.fori_loop(0, (NCHUNK - 10) // 12, body, 0)
    for i in range(NCHUNK - 10, NCHUNK):
        chunk(i, i % 4, i % 6)
    scat_wait((NCHUNK - 1) % 4)
    plsc.subcore_barrier()

    @pl.when(s < NS - 1)
    def _():
        o = pl.multiple_of(s * RTA, 8)
        pltpu.sync_copy(acc.at[pl.ds(o, RTA)], out_hbm.at[c, pl.ds(o, RTA)])
    @pl.when(s == NS - 1)
    def _():
        pltpu.sync_copy(acc.at[pl.ds((NS - 1) * RTA, RTB)],
                        out_hbm.at[c, pl.ds((NS - 1) * RTA, RTB)])


def _scatter_call(x, src3, dst3, zeros2d):
    k = pl.kernel(
        _scatter_body,
        out_type=jax.ShapeDtypeStruct((NC, N, F), jnp.float32),
        mesh=_MESH,
        scratch_types=[
            pltpu.VMEM((6, CH), jnp.int32),
            pltpu.VMEM((6, CH), jnp.int32),
            pltpu.VMEM((4, CH, F), jnp.float32),
            pltpu.VMEM_SHARED((N, F), jnp.float32),
            pltpu.SemaphoreType.DMA((6,)),
            pltpu.SemaphoreType.DMA((4,)),
            pltpu.SemaphoreType.DMA((4,)),
        ],
    )
    return k(x, src3, dst3, zeros2d)


_R = 2000
_G = N // _R


def _tc1_body(x_ref, w1_ref, b1_ref, w2_ref, b2_ref, dinv_ref, h_ref, u_ref):
    x = x_ref[...]
    h = jnp.maximum(jnp.dot(x, w1_ref[...],
                            preferred_element_type=jnp.float32) + b1_ref[...], 0.0)
    h = jnp.maximum(jnp.dot(h, w2_ref[...],
                            preferred_element_type=jnp.float32) + b2_ref[...], 0.0)
    h_ref[...] = h
    u_ref[...] = h * dinv_ref[...]


def _tc1_call(x, W1, b1, W2, b2, dinvc):
    full = lambda shp: pl.BlockSpec(shp, lambda i: (0,) * len(shp))
    return pl.pallas_call(
        _tc1_body,
        grid=(_G,),
        in_specs=[
            pl.BlockSpec((_R, F), lambda i: (i, 0)),
            full((F, F)), full((F,)), full((F, F)), full((F,)),
            pl.BlockSpec((_R, 1), lambda i: (i, 0)),
        ],
        out_specs=[pl.BlockSpec((_R, F), lambda i: (i, 0))] * 2,
        out_shape=[jax.ShapeDtypeStruct((N, F), jnp.float32)] * 2,
    )(x, W1, b1, W2, b2, dinvc)


def _tc2_body(s1_ref, dinv_ref, dinv2_ref, b_ref, v_ref):
    s1 = s1_ref[0] + s1_ref[1]
    b_ref[...] = s1 * dinv_ref[...]
    v_ref[...] = s1 * dinv2_ref[...]


def _tc2_call(s1p, dinvc, dinv2c):
    return pl.pallas_call(
        _tc2_body,
        grid=(_G,),
        in_specs=[
            pl.BlockSpec((NC, _R, F), lambda i: (0, i, 0)),
            pl.BlockSpec((_R, 1), lambda i: (i, 0)),
            pl.BlockSpec((_R, 1), lambda i: (i, 0)),
        ],
        out_specs=[pl.BlockSpec((_R, F), lambda i: (i, 0))] * 2,
        out_shape=[jax.ShapeDtypeStruct((N, F), jnp.float32)] * 2,
    )(s1p, dinvc, dinv2c)


def _tc3_body(h_ref, b_ref, s2_ref, dinv_ref, diag1_ref, wl1_ref, bl1_ref,
              bl2_ref, bl3_ref, diag2_ref, diag3_ref, w3_ref, b3_ref,
              w4_ref, b4_ref, logits_ref, hh_ref):
    h = h_ref[...]
    B = b_ref[...]
    C = (s2_ref[0] + s2_ref[1]) * dinv_ref[...]
    diag1 = diag1_ref[...]
    diag2 = diag2_ref[...]
    diag3 = diag3_ref[...]

    def f12(dg):
        d1, d2 = dg[1], dg[2]
        f1 = h - B * d1
        f2 = h - B * (d1 + d2) + C * (d1 * d2)
        return f1, f2

    f1_1, f2_1 = f12(diag1)
    h1 = (jnp.dot(3.0 * h * diag1[0], wl1_ref[...],
                  preferred_element_type=jnp.float32) + bl1_ref[...]
          - 3.0 * f1_1 + 0.75 * f2_1)
    f1_2, f2_2 = f12(diag2)
    h2 = bl2_ref[...] + 3.0 * f1_2 - 1.5 * f2_2
    _, f2_3 = f12(diag3)
    h3 = bl3_ref[...] + 0.75 * f2_3

    w3 = w3_ref[...]
    hh = (jnp.dot(h1, w3[0:F], preferred_element_type=jnp.float32)
          + jnp.dot(h2, w3[F:2 * F], preferred_element_type=jnp.float32)
          + jnp.dot(h3, w3[2 * F:3 * F], preferred_element_type=jnp.float32)
          + b3_ref[...])
    hh = jnp.maximum(hh, 0.0)
    hh_ref[...] = hh
    logits_ref[...] = jnp.dot(hh, w4_ref[...],
                              preferred_element_type=jnp.float32) + b4_ref[...]


def _tc3_call(h, Bmat, s2p, dinvc, diag1, Wl1, bl1, bl2, bl3, diag2, diag3,
              W3, b3, W4, b4):
    full = lambda shp: pl.BlockSpec(shp, lambda i: (0,) * len(shp))
    return pl.pallas_call(
        _tc3_body,
        grid=(_G,),
        in_specs=[
            pl.BlockSpec((_R, F), lambda i: (i, 0)),
            pl.BlockSpec((_R, F), lambda i: (i, 0)),
            pl.BlockSpec((NC, _R, F), lambda i: (0, i, 0)),
            pl.BlockSpec((_R, 1), lambda i: (i, 0)),
            full((3, F)), full((F, F)), full((F,)), full((F,)), full((F,)),
            full((3, F)), full((3, F)), full((3 * F, F)), full((F,)),
            full((F, 2)), full((2,)),
        ],
        out_specs=[pl.BlockSpec((_R, 2), lambda i: (i, 0)),
                   pl.BlockSpec((_R, F), lambda i: (i, 0))],
        out_shape=[jax.ShapeDtypeStruct((N, 2), jnp.float32),
                   jax.ShapeDtypeStruct((N, F), jnp.float32)],
    )(h, Bmat, s2p, dinvc, diag1, Wl1, bl1, bl2, bl3, diag2, diag3,
      W3, b3, W4, b4)


def kernel(in_feat, edge_index, W1, b1, W2, b2, diag1, Wl1, bl1,
           diag2, Wl2, bl2, diag3, Wl3, bl3, W3, b3, W4, b4):
    src3 = edge_index[0].reshape(NW, NCHUNK, CH)
    dst3 = edge_index[1].reshape(NW, NCHUNK, CH)
    zeros1d = jnp.zeros((N,), jnp.float32)
    zeros2d = jnp.zeros((N, F), jnp.float32)

    degp = _deg_call(dst3, zeros1d)
    deg = degp[0] + degp[1]
    dinv = lax.rsqrt(jnp.maximum(deg, 1.0))
    dinvc = dinv[:, None]
    dinv2c = (dinv * dinv)[:, None]

    h, u = _tc1_call(in_feat, W1, b1, W2, b2, dinvc)
    s1p = _scatter_call(u, src3, dst3, zeros2d)
    Bmat, v = _tc2_call(s1p, dinvc, dinv2c)
    s2p = _scatter_call(v, src3, dst3, zeros2d)
    logits, hh = _tc3_call(h, Bmat, s2p, dinvc, diag1, Wl1, bl1, bl2, bl3,
                           diag2, diag3, W3, b3, W4, b4)
    return (logits, hh)


# speedup vs baseline: 1.7446x; 1.0487x over previous
import functools

import jax
import jax.numpy as jnp
from jax import lax
from jax.experimental import pallas as pl
from jax.experimental.pallas import tpu as pltpu
from jax.experimental.pallas import tpu_sc as plsc

N = 10000
E = 320000
F = 128
NC = 2
NS = 16
NW = NC * NS
EW = E // NW
CH = 40
NCHUNK = EW // CH
RTA = 624
RTB = N - (NS - 1) * RTA

_MESH = plsc.VectorSubcoreMesh(
    core_axis_name="c", subcore_axis_name="s", num_cores=NC, num_subcores=NS)


def _deg_body(dst3_hbm, zeros_hbm, out_hbm, didx, ones_v, acc, isem, ssem):
    c = lax.axis_index("c")
    s = lax.axis_index("s")
    wid = s * NC + c
    for off in (0, 16, 24):
        ones_v[pl.ds(off, 16)] = jnp.ones((16,), jnp.float32)
    @pl.when(s == 0)
    def _():
        pltpu.sync_copy(zeros_hbm, acc)

    def idx_start(i, q):
        pltpu.async_copy(dst3_hbm.at[wid, i], didx.at[q], isem.at[q])

    def idx_wait(i, q):
        pltpu.make_async_copy(dst3_hbm.at[wid, i], didx.at[q],
                              isem.at[q]).wait()

    def scat_wait(q):
        pltpu.make_async_copy(ones_v, acc.at[didx.at[q]], ssem.at[q]).wait()

    def chunk(i, q):
        if isinstance(i, int):
            if i >= 4:
                scat_wait((q + 4) % 8)
            if i + 4 < NCHUNK:
                idx_start(i + 4, (q + 4) % 8)
        else:
            @pl.when(i >= 4)
            def _():
                scat_wait((q + 4) % 8)
            idx_start(i + 4, (q + 4) % 8)
        idx_wait(i, q)
        pltpu.async_copy(ones_v, acc.at[didx.at[q]], ssem.at[q], add=True)

    for j in range(4):
        idx_start(j, j)
    plsc.subcore_barrier()

    def body(t, carry):
        for b in range(8):
            chunk(8 * t + b, b)
        return carry
    lax.fori_loop(0, (NCHUNK - 10) // 8, body, 0)
    for i in range(NCHUNK - 10, NCHUNK):
        chunk(i, i % 8)
    for i in range(NCHUNK - 4, NCHUNK):
        scat_wait(i % 8)
    plsc.subcore_barrier()

    @pl.when(s == 0)
    def _():
        pltpu.sync_copy(acc, out_hbm.at[c])


def _deg_call(dst3, zeros1d):
    k = pl.kernel(
        _deg_body,
        out_type=jax.ShapeDtypeStruct((NC, N), jnp.float32),
        mesh=_MESH,
        scratch_types=[
            pltpu.VMEM((8, CH), jnp.int32),
            pltpu.VMEM((CH,), jnp.float32),
            pltpu.VMEM_SHARED((N,), jnp.float32),
            pltpu.SemaphoreType.DMA((8,)),
            pltpu.SemaphoreType.DMA((8,)),
        ],
    )
    return k(dst3, zeros1d)


NBUF = 2
NB = NCHUNK // NBUF


def _scatter_body(x_hbm, src3_hbm, dst3_hbm, zeros_hbm, out_hbm,
                  sidx, didx, rows, acc, isem, gsem, ssem):
    c = lax.axis_index("c")
    s = lax.axis_index("s")
    wid = s * NC + c
    @pl.when(s < NS - 1)
    def _():
        o = pl.multiple_of(s * RTA, 8)
        pltpu.sync_copy(zeros_hbm.at[pl.ds(o, RTA)], acc.at[pl.ds(o, RTA)])
    @pl.when(s == NS - 1)
    def _():
        pltpu.sync_copy(zeros_hbm.at[pl.ds((NS - 1) * RTA, RTB)],
                        acc.at[pl.ds((NS - 1) * RTA, RTB)])

    def idx_start(i, q):
        pltpu.async_copy(src3_hbm.at[wid, i], sidx.at[q], isem.at[q])
        pltpu.async_copy(dst3_hbm.at[wid, i], didx.at[q], isem.at[q])

    def idx_wait(i, q):
        pltpu.make_async_copy(src3_hbm.at[wid, i], sidx.at[q],
                              isem.at[q]).wait()
        pltpu.make_async_copy(dst3_hbm.at[wid, i], didx.at[q],
                              isem.at[q]).wait()

    def gat_start(i, r, q):
        pltpu.async_copy(x_hbm.at[sidx.at[q]], rows.at[r], gsem.at[r])

    def gat_wait(r, q):
        pltpu.make_async_copy(x_hbm.at[sidx.at[q]], rows.at[r],
                              gsem.at[r]).wait()

    def scat_wait(r):
        pltpu.make_async_copy(rows.at[r], acc.at[didx.at[0]],
                              ssem.at[r]).wait()

    def chunk(i, r, q):
        gat_wait(r, q)
        pltpu.async_copy(rows.at[r], acc.at[didx.at[q]], ssem.at[r],
                         add=True)
        if isinstance(i, int):
            if i >= 1:
                scat_wait((r + 3) % 4)
            if i + 5 < NCHUNK:
                idx_start(i + 5, (q + 5) % 6)
            if i + 3 < NCHUNK:
                idx_wait(i + 3, (q + 3) % 6)
                gat_start(i + 3, (r + 3) % 4, (q + 3) % 6)
        else:
            @pl.when(i >= 1)
            def _():
                scat_wait((r + 3) % 4)
            idx_start(i + 5, (q + 5) % 6)
            idx_wait(i + 3, (q + 3) % 6)
            gat_start(i + 3, (r + 3) % 4, (q + 3) % 6)

    for j in range(5):
        idx_start(j, j)
    plsc.subcore_barrier()
    idx_wait(0, 0)
    gat_start(0, 0, 0)
    idx_wait(1, 1)
    gat_start(1, 1, 1)
    idx_wait(2, 2)
    gat_start(2, 2, 2)

    def body(t, carry):
        for b in range(12):
            chunk(12 * t + b, b % 4, b % 6)
        return carry
    lax.fori_loop(0, (NCHUNK - 10) // 12, body, 0)
    for i in range(NCHUNK - 10, NCHUNK):
        chunk(i, i % 4, i % 6)
    scat_wait((NCHUNK - 1) % 4)
    plsc.subcore_barrier()

    @pl.when(s < NS - 1)
    def _():
        o = pl.multiple_of(s * RTA, 8)
        pltpu.sync_copy(acc.at[pl.ds(o, RTA)], out_hbm.at[c, pl.ds(o, RTA)])
    @pl.when(s == NS - 1)
    def _():
        pltpu.sync_copy(acc.at[pl.ds((NS - 1) * RTA, RTB)],
                        out_hbm.at[c, pl.ds((NS - 1) * RTA, RTB)])


def _scatter_call(x, src3, dst3, zeros2d):
    k = pl.kernel(
        _scatter_body,
        out_type=jax.ShapeDtypeStruct((NC, N, F), jnp.float32),
        mesh=_MESH,
        scratch_types=[
            pltpu.VMEM((6, CH), jnp.int32),
            pltpu.VMEM((6, CH), jnp.int32),
            pltpu.VMEM((4, CH, F), jnp.float32),
            pltpu.VMEM_SHARED((N, F), jnp.float32),
            pltpu.SemaphoreType.DMA((6,)),
            pltpu.SemaphoreType.DMA((4,)),
            pltpu.SemaphoreType.DMA((4,)),
        ],
    )
    return k(x, src3, dst3, zeros2d)


_R = 2000
_G = N // _R


def _tc1_body(x_ref, w1_ref, b1_ref, w2_ref, b2_ref, dinv_ref, h_ref, u_ref):
    x = x_ref[...]
    h = jnp.maximum(jnp.dot(x, w1_ref[...],
                            preferred_element_type=jnp.float32) + b1_ref[...], 0.0)
    h = jnp.maximum(jnp.dot(h, w2_ref[...],
                            preferred_element_type=jnp.float32) + b2_ref[...], 0.0)
    h_ref[...] = h
    u_ref[...] = h * dinv_ref[...]


def _tc1_call(x, W1, b1, W2, b2, dinvc):
    full = lambda shp: pl.BlockSpec(shp, lambda i: (0,) * len(shp))
    return pl.pallas_call(
        _tc1_body,
        grid=(_G,),
        in_specs=[
            pl.BlockSpec((_R, F), lambda i: (i, 0)),
            full((F, F)), full((F,)), full((F, F)), full((F,)),
            pl.BlockSpec((_R, 1), lambda i: (i, 0)),
        ],
        out_specs=[pl.BlockSpec((_R, F), lambda i: (i, 0))] * 2,
        out_shape=[jax.ShapeDtypeStruct((N, F), jnp.float32)] * 2,
    )(x, W1, b1, W2, b2, dinvc)


def _tc2_body(s1_ref, dinv_ref, dinv2_ref, b_ref, v_ref):
    s1 = s1_ref[0] + s1_ref[1]
    b_ref[...] = s1 * dinv_ref[...]
    v_ref[...] = s1 * dinv2_ref[...]


def _tc2_call(s1p, dinvc, dinv2c):
    return pl.pallas_call(
        _tc2_body,
        grid=(_G,),
        in_specs=[
            pl.BlockSpec((NC, _R, F), lambda i: (0, i, 0)),
            pl.BlockSpec((_R, 1), lambda i: (i, 0)),
            pl.BlockSpec((_R, 1), lambda i: (i, 0)),
        ],
        out_specs=[pl.BlockSpec((_R, F), lambda i: (i, 0))] * 2,
        out_shape=[jax.ShapeDtypeStruct((N, F), jnp.float32)] * 2,
    )(s1p, dinvc, dinv2c)


def _tc3_body(h_ref, b_ref, s2_ref, dinv_ref, diag1_ref, wl1_ref, bl1_ref,
              bl2_ref, bl3_ref, diag2_ref, diag3_ref, w3_ref, b3_ref,
              w4_ref, b4_ref, logits_ref, hh_ref):
    h = h_ref[...]
    B = b_ref[...]
    C = (s2_ref[0] + s2_ref[1]) * dinv_ref[...]
    diag1 = diag1_ref[...]
    diag2 = diag2_ref[...]
    diag3 = diag3_ref[...]

    def f12(dg):
        d1, d2 = dg[1], dg[2]
        f1 = h - B * d1
        f2 = h - B * (d1 + d2) + C * (d1 * d2)
        return f1, f2

    f1_1, f2_1 = f12(diag1)
    h1 = (jnp.dot(3.0 * h * diag1[0], wl1_ref[...],
                  preferred_element_type=jnp.float32) + bl1_ref[...]
          - 3.0 * f1_1 + 0.75 * f2_1)
    f1_2, f2_2 = f12(diag2)
    h2 = bl2_ref[...] + 3.0 * f1_2 - 1.5 * f2_2
    _, f2_3 = f12(diag3)
    h3 = bl3_ref[...] + 0.75 * f2_3

    w3 = w3_ref[...]
    hh = (jnp.dot(h1, w3[0:F], preferred_element_type=jnp.float32)
          + jnp.dot(h2, w3[F:2 * F], preferred_element_type=jnp.float32)
          + jnp.dot(h3, w3[2 * F:3 * F], preferred_element_type=jnp.float32)
          + b3_ref[...])
    hh = jnp.maximum(hh, 0.0)
    hh_ref[...] = hh
    logits_ref[...] = jnp.dot(hh, w4_ref[...],
                              preferred_element_type=jnp.float32) + b4_ref[...]


def _tc3_call(h, Bmat, s2p, dinvc, diag1, Wl1, bl1, bl2, bl3, diag2, diag3,
              W3, b3, W4, b4):
    full = lambda shp: pl.BlockSpec(shp, lambda i: (0,) * len(shp))
    return pl.pallas_call(
        _tc3_body,
        grid=(_G,),
        in_specs=[
            pl.BlockSpec((_R, F), lambda i: (i, 0)),
            pl.BlockSpec((_R, F), lambda i: (i, 0)),
            pl.BlockSpec((NC, _R, F), lambda i: (0, i, 0)),
            pl.BlockSpec((_R, 1), lambda i: (i, 0)),
            full((3, F)), full((F, F)), full((F,)), full((F,)), full((F,)),
            full((3, F)), full((3, F)), full((3 * F, F)), full((F,)),
            full((F, 2)), full((2,)),
        ],
        out_specs=[pl.BlockSpec((_R, 2), lambda i: (i, 0)),
                   pl.BlockSpec((_R, F), lambda i: (i, 0))],
        out_shape=[jax.ShapeDtypeStruct((N, 2), jnp.float32),
                   jax.ShapeDtypeStruct((N, F), jnp.float32)],
    )(h, Bmat, s2p, dinvc, diag1, Wl1, bl1, bl2, bl3, diag2, diag3,
      W3, b3, W4, b4)


def kernel(in_feat, edge_index, W1, b1, W2, b2, diag1, Wl1, bl1,
           diag2, Wl2, bl2, diag3, Wl3, bl3, W3, b3, W4, b4):
    src3 = edge_index[0].reshape(NW, NCHUNK, CH)
    dst3 = edge_index[1].reshape(NW, NCHUNK, CH)
    zeros1d = jnp.zeros((N,), jnp.float32)
    zeros2d = jnp.zeros((N, F), jnp.float32)

    degp = _deg_call(dst3, zeros1d)
    deg = degp[0] + degp[1]
    dinv = lax.rsqrt(jnp.maximum(deg, 1.0))
    dinvc = dinv[:, None]
    dinv2c = (dinv * dinv)[:, None]

    h, u = _tc1_call(in_feat, W1, b1, W2, b2, dinvc)
    s1p = _scatter_call(u, src3, dst3, zeros2d)
    Bmat, v = _tc2_call(s1p, dinvc, dinv2c)
    s2p = _scatter_call(v, src3, dst3, zeros2d)
    logits, hh = _tc3_call(h, Bmat, s2p, dinvc, diag1, Wl1, bl1, bl2, bl3,
                           diag2, diag3, W3, b3, W4, b4)
    return (logits, hh)


# speedup vs baseline: 1.7498x; 1.0030x over previous
import functools

import jax
import jax.numpy as jnp
from jax import lax
from jax.experimental import pallas as pl
from jax.experimental.pallas import tpu as pltpu
from jax.experimental.pallas import tpu_sc as plsc

N = 10000
E = 320000
F = 128
NC = 2
NS = 16
NW = NC * NS
EW = E // NW
CH = 40
NCHUNK = EW // CH
RTA = 624
RTB = N - (NS - 1) * RTA

_MESH = plsc.VectorSubcoreMesh(
    core_axis_name="c", subcore_axis_name="s", num_cores=NC, num_subcores=NS)


def _deg_body(dst3_hbm, zeros_hbm, out_hbm, didx, ones_v, acc, isem, ssem):
    c = lax.axis_index("c")
    s = lax.axis_index("s")
    wid = s * NC + c
    for off in (0, 16, 24):
        ones_v[pl.ds(off, 16)] = jnp.ones((16,), jnp.float32)
    @pl.when(s == 0)
    def _():
        pltpu.sync_copy(zeros_hbm, acc)

    def idx_start(i, q):
        pltpu.async_copy(dst3_hbm.at[wid, i], didx.at[q], isem.at[q])

    def idx_wait(i, q):
        pltpu.make_async_copy(dst3_hbm.at[wid, i], didx.at[q],
                              isem.at[q]).wait()

    def scat_wait(q):
        pltpu.make_async_copy(ones_v, acc.at[didx.at[q]], ssem.at[q]).wait()

    def chunk(i, q):
        if isinstance(i, int):
            if i >= 4:
                scat_wait((q + 4) % 8)
            if i + 4 < NCHUNK:
                idx_start(i + 4, (q + 4) % 8)
        else:
            @pl.when(i >= 4)
            def _():
                scat_wait((q + 4) % 8)
            idx_start(i + 4, (q + 4) % 8)
        idx_wait(i, q)
        pltpu.async_copy(ones_v, acc.at[didx.at[q]], ssem.at[q], add=True)

    for j in range(4):
        idx_start(j, j)
    plsc.subcore_barrier()

    def body(t, carry):
        for b in range(8):
            chunk(8 * t + b, b)
        return carry
    lax.fori_loop(0, (NCHUNK - 10) // 8, body, 0)
    for i in range(NCHUNK - 10, NCHUNK):
        chunk(i, i % 8)
    for i in range(NCHUNK - 4, NCHUNK):
        scat_wait(i % 8)
    plsc.subcore_barrier()

    @pl.when(s == 0)
    def _():
        pltpu.sync_copy(acc, out_hbm.at[c])


def _deg_call(dst3, zeros1d):
    k = pl.kernel(
        _deg_body,
        out_type=jax.ShapeDtypeStruct((NC, N), jnp.float32),
        mesh=_MESH,
        scratch_types=[
            pltpu.VMEM((8, CH), jnp.int32),
            pltpu.VMEM((CH,), jnp.float32),
            pltpu.VMEM_SHARED((N,), jnp.float32),
            pltpu.SemaphoreType.DMA((8,)),
            pltpu.SemaphoreType.DMA((8,)),
        ],
    )
    return k(dst3, zeros1d)


NBUF = 2
NB = NCHUNK // NBUF


def _scatter_body(x_hbm, src3_hbm, dst3_hbm, zeros_hbm, out_hbm,
                  sidx, didx, rows, acc, isem, gsem, ssem):
    c = lax.axis_index("c")
    s = lax.axis_index("s")
    wid = s * NC + c
    @pl.when(s < NS - 1)
    def _():
        o = pl.multiple_of(s * RTA, 8)
        pltpu.sync_copy(zeros_hbm.at[pl.ds(o, RTA)], acc.at[pl.ds(o, RTA)])
    @pl.when(s == NS - 1)
    def _():
        pltpu.sync_copy(zeros_hbm.at[pl.ds((NS - 1) * RTA, RTB)],
                        acc.at[pl.ds((NS - 1) * RTA, RTB)])

    def idx_start(i, q):
        pltpu.async_copy(src3_hbm.at[wid, i], sidx.at[q], isem.at[q])
        pltpu.async_copy(dst3_hbm.at[wid, i], didx.at[q], isem.at[q])

    def idx_wait(i, q):
        pltpu.make_async_copy(src3_hbm.at[wid, i], sidx.at[q],
                              isem.at[q]).wait()
        pltpu.make_async_copy(dst3_hbm.at[wid, i], didx.at[q],
                              isem.at[q]).wait()

    def gat_start(i, r, q):
        pltpu.async_copy(x_hbm.at[sidx.at[q]], rows.at[r], gsem.at[r])

    def gat_wait(r, q):
        pltpu.make_async_copy(x_hbm.at[sidx.at[q]], rows.at[r],
                              gsem.at[r]).wait()

    def scat_wait(r):
        pltpu.make_async_copy(rows.at[r], acc.at[didx.at[0]],
                              ssem.at[r]).wait()

    def chunk(i, r, q):
        gat_wait(r, q)
        pltpu.async_copy(rows.at[r], acc.at[didx.at[q]], ssem.at[r],
                         add=True)
        if isinstance(i, int):
            if i >= 1:
                scat_wait((r + 3) % 4)
            if i + 5 < NCHUNK:
                idx_start(i + 5, (q + 5) % 6)
            if i + 3 < NCHUNK:
                idx_wait(i + 3, (q + 3) % 6)
                gat_start(i + 3, (r + 3) % 4, (q + 3) % 6)
        else:
            @pl.when(i >= 1)
            def _():
                scat_wait((r + 3) % 4)
            idx_start(i + 5, (q + 5) % 6)
            idx_wait(i + 3, (q + 3) % 6)
            gat_start(i + 3, (r + 3) % 4, (q + 3) % 6)

    for j in range(5):
        idx_start(j, j)
    plsc.subcore_barrier()
    idx_wait(0, 0)
    gat_start(0, 0, 0)
    idx_wait(1, 1)
    gat_start(1, 1, 1)
    idx_wait(2, 2)
    gat_start(2, 2, 2)

    def body(t, carry):
        for b in range(12):
            chunk(12 * t + b, b % 4, b % 6)
        return carry
    lax.fori_loop(0, (NCHUNK - 10) // 12, body, 0)
    for i in range(NCHUNK - 10, NCHUNK):
        chunk(i, i % 4, i % 6)
    scat_wait((NCHUNK - 1) % 4)
    plsc.subcore_barrier()

    @pl.when(s < NS - 1)
    def _():
        o = pl.multiple_of(s * RTA, 8)
        pltpu.sync_copy(acc.at[pl.ds(o, RTA)], out_hbm.at[c, pl.ds(o, RTA)])
    @pl.when(s == NS - 1)
    def _():
        pltpu.sync_copy(acc.at[pl.ds((NS - 1) * RTA, RTB)],
                        out_hbm.at[c, pl.ds((NS - 1) * RTA, RTB)])


def _scatter_call(x, src3, dst3, zeros2d):
    k = pl.kernel(
        _scatter_body,
        out_type=jax.ShapeDtypeStruct((NC, N, F), jnp.float32),
        mesh=_MESH,
        scratch_types=[
            pltpu.VMEM((6, CH), jnp.int32),
            pltpu.VMEM((6, CH), jnp.int32),
            pltpu.VMEM((4, CH, F), jnp.float32),
            pltpu.VMEM_SHARED((N, F), jnp.float32),
            pltpu.SemaphoreType.DMA((6,)),
            pltpu.SemaphoreType.DMA((4,)),
            pltpu.SemaphoreType.DMA((4,)),
        ],
    )
    return k(x, src3, dst3, zeros2d)


_R = 2000
_G = N // _R


def _tc1a_body(x_ref, w1_ref, b1_ref, w2_ref, b2_ref, h_ref):
    x = x_ref[...]
    h = jnp.maximum(jnp.dot(x, w1_ref[...],
                            preferred_element_type=jnp.float32) + b1_ref[...], 0.0)
    h = jnp.maximum(jnp.dot(h, w2_ref[...],
                            preferred_element_type=jnp.float32) + b2_ref[...], 0.0)
    h_ref[...] = h


def _tc1a_call(x, W1, b1, W2, b2):
    full = lambda shp: pl.BlockSpec(shp, lambda i: (0,) * len(shp))
    return pl.pallas_call(
        _tc1a_body,
        grid=(_G,),
        in_specs=[
            pl.BlockSpec((_R, F), lambda i: (i, 0)),
            full((F, F)), full((F,)), full((F, F)), full((F,)),
        ],
        out_specs=pl.BlockSpec((_R, F), lambda i: (i, 0)),
        out_shape=jax.ShapeDtypeStruct((N, F), jnp.float32),
    )(x, W1, b1, W2, b2)


def _tc1b_body(h_ref, dinv_ref, u_ref):
    u_ref[...] = h_ref[...] * dinv_ref[...]


def _tc1b_call(h, dinvc):
    return pl.pallas_call(
        _tc1b_body,
        grid=(_G,),
        in_specs=[
            pl.BlockSpec((_R, F), lambda i: (i, 0)),
            pl.BlockSpec((_R, 1), lambda i: (i, 0)),
        ],
        out_specs=pl.BlockSpec((_R, F), lambda i: (i, 0)),
        out_shape=jax.ShapeDtypeStruct((N, F), jnp.float32),
    )(h, dinvc)


def _tc2_body(s1_ref, dinv_ref, dinv2_ref, b_ref, v_ref):
    s1 = s1_ref[0] + s1_ref[1]
    b_ref[...] = s1 * dinv_ref[...]
    v_ref[...] = s1 * dinv2_ref[...]


def _tc2_call(s1p, dinvc, dinv2c):
    return pl.pallas_call(
        _tc2_body,
        grid=(_G,),
        in_specs=[
            pl.BlockSpec((NC, _R, F), lambda i: (0, i, 0)),
            pl.BlockSpec((_R, 1), lambda i: (i, 0)),
            pl.BlockSpec((_R, 1), lambda i: (i, 0)),
        ],
        out_specs=[pl.BlockSpec((_R, F), lambda i: (i, 0))] * 2,
        out_shape=[jax.ShapeDtypeStruct((N, F), jnp.float32)] * 2,
    )(s1p, dinvc, dinv2c)


def _tc3_body(h_ref, b_ref, s2_ref, dinv_ref, diag1_ref, wl1_ref, bl1_ref,
              bl2_ref, bl3_ref, diag2_ref, diag3_ref, w3_ref, b3_ref,
              w4_ref, b4_ref, logits_ref, hh_ref):
    h = h_ref[...]
    B = b_ref[...]
    C = (s2_ref[0] + s2_ref[1]) * dinv_ref[...]
    diag1 = diag1_ref[...]
    diag2 = diag2_ref[...]
    diag3 = diag3_ref[...]

    def f12(dg):
        d1, d2 = dg[1], dg[2]
        f1 = h - B * d1
        f2 = h - B * (d1 + d2) + C * (d1 * d2)
        return f1, f2

    f1_1, f2_1 = f12(diag1)
    h1 = (jnp.dot(3.0 * h * diag1[0], wl1_ref[...],
                  preferred_element_type=jnp.float32) + bl1_ref[...]
          - 3.0 * f1_1 + 0.75 * f2_1)
    f1_2, f2_2 = f12(diag2)
    h2 = bl2_ref[...] + 3.0 * f1_2 - 1.5 * f2_2
    _, f2_3 = f12(diag3)
    h3 = bl3_ref[...] + 0.75 * f2_3

    w3 = w3_ref[...]
    hh = (jnp.dot(h1, w3[0:F], preferred_element_type=jnp.float32)
          + jnp.dot(h2, w3[F:2 * F], preferred_element_type=jnp.float32)
          + jnp.dot(h3, w3[2 * F:3 * F], preferred_element_type=jnp.float32)
          + b3_ref[...])
    hh = jnp.maximum(hh, 0.0)
    hh_ref[...] = hh
    logits_ref[...] = jnp.dot(hh, w4_ref[...],
                              preferred_element_type=jnp.float32) + b4_ref[...]


def _tc3_call(h, Bmat, s2p, dinvc, diag1, Wl1, bl1, bl2, bl3, diag2, diag3,
              W3, b3, W4, b4):
    full = lambda shp: pl.BlockSpec(shp, lambda i: (0,) * len(shp))
    return pl.pallas_call(
        _tc3_body,
        grid=(_G,),
        in_specs=[
            pl.BlockSpec((_R, F), lambda i: (i, 0)),
            pl.BlockSpec((_R, F), lambda i: (i, 0)),
            pl.BlockSpec((NC, _R, F), lambda i: (0, i, 0)),
            pl.BlockSpec((_R, 1), lambda i: (i, 0)),
            full((3, F)), full((F, F)), full((F,)), full((F,)), full((F,)),
            full((3, F)), full((3, F)), full((3 * F, F)), full((F,)),
            full((F, 2)), full((2,)),
        ],
        out_specs=[pl.BlockSpec((_R, 2), lambda i: (i, 0)),
                   pl.BlockSpec((_R, F), lambda i: (i, 0))],
        out_shape=[jax.ShapeDtypeStruct((N, 2), jnp.float32),
                   jax.ShapeDtypeStruct((N, F), jnp.float32)],
    )(h, Bmat, s2p, dinvc, diag1, Wl1, bl1, bl2, bl3, diag2, diag3,
      W3, b3, W4, b4)


def kernel(in_feat, edge_index, W1, b1, W2, b2, diag1, Wl1, bl1,
           diag2, Wl2, bl2, diag3, Wl3, bl3, W3, b3, W4, b4):
    src3 = edge_index[0].reshape(NW, NCHUNK, CH)
    dst3 = edge_index[1].reshape(NW, NCHUNK, CH)
    zeros1d = jnp.zeros((N,), jnp.float32)
    zeros2d = jnp.zeros((N, F), jnp.float32)

    degp = _deg_call(dst3, zeros1d)
    h = _tc1a_call(in_feat, W1, b1, W2, b2)
    deg = degp[0] + degp[1]
    dinv = lax.rsqrt(jnp.maximum(deg, 1.0))
    dinvc = dinv[:, None]
    dinv2c = (dinv * dinv)[:, None]

    u = _tc1b_call(h, dinvc)
    s1p = _scatter_call(u, src3, dst3, zeros2d)
    Bmat, v = _tc2_call(s1p, dinvc, dinv2c)
    s2p = _scatter_call(v, src3, dst3, zeros2d)
    logits, hh = _tc3_call(h, Bmat, s2p, dinvc, diag1, Wl1, bl1, bl2, bl3,
                           diag2, diag3, W3, b3, W4, b4)
    return (logits, hh)
